# Initial kernel scaffold; baseline (speedup 1.0000x reference)
import jax
import jax.numpy as jnp
from jax.experimental import pallas as pl


def kernel(x):
    raise NotImplementedError("write your pallas kernel here")


# speedup vs baseline: 6.8158x; 6.8158x over previous
import functools

import jax
import jax.numpy as jnp
from jax.experimental import pallas as pl

_FACTOR = 6.26
_TOPK = 128


def _kcomp_block(x_ref, o_ref, *, k1, k2, factor, idx_bits):
    x = x_ref[...]
    r, d = x.shape
    pos = jnp.maximum(x, 0.0)
    neg = jnp.maximum(-x, 0.0)
    pos_bits = jax.lax.bitcast_convert_type(pos, jnp.int32) & 0x7FFFFFFF
    neg_bits = jax.lax.bitcast_convert_type(neg, jnp.int32) & 0x7FFFFFFF

    def count_ge(bits, thr):
        return jnp.sum((bits >= thr).astype(jnp.int32), axis=1, keepdims=True)

    def val_step(i, carry):
        tp, tn = carry
        bit = jnp.int32(1) << (30 - i)
        candp = tp | bit
        candn = tn | bit
        tp = jnp.where(count_ge(pos_bits, candp) >= k1, candp, tp)
        tn = jnp.where(count_ge(neg_bits, candn) >= k2, candn, tn)
        return tp, tn

    zeros = jnp.zeros((r, 1), jnp.int32)
    tp, tn = jax.lax.fori_loop(0, 31, val_step, (zeros, zeros))

    cnt_gt_p = count_ge(pos_bits, tp + 1)
    cnt_gt_n = count_ge(neg_bits, tn + 1)
    need_p = k1 - cnt_gt_p
    need_n = k2 - cnt_gt_n

    key = (d - 1) - jax.lax.broadcasted_iota(jnp.int32, (r, d), 1)
    eq_p = pos_bits == tp
    eq_n = neg_bits == tn

    def idx_step(i, carry):
        kp, kn = carry
        bit = jnp.int32(1) << (idx_bits - 1 - i)
        candp = kp | bit
        candn = kn | bit
        cp = jnp.sum((eq_p & (key >= candp)).astype(jnp.int32), axis=1,
                     keepdims=True)
        cn = jnp.sum((eq_n & (key >= candn)).astype(jnp.int32), axis=1,
                     keepdims=True)
        kp = jnp.where(cp >= need_p, candp, kp)
        kn = jnp.where(cn >= need_n, candn, kn)
        return kp, kn

    kp, kn = jax.lax.fori_loop(0, idx_bits, idx_step, (zeros, zeros))

    maskp = (pos_bits > tp) | (eq_p & (key >= kp))
    maskn = (neg_bits > tn) | (eq_n & (key >= kn))

    loser_p = jnp.sum(jnp.where(maskp, 0.0, pos), axis=1, keepdims=True)
    loser_n = jnp.sum(jnp.where(maskn, 0.0, neg), axis=1, keepdims=True)
    ptmp = factor * loser_p
    ntmp = factor * loser_n

    o_ref[...] = (jnp.where(maskp, pos + ptmp, 0.0)
                  - jnp.where(maskn, neg + ntmp, 0.0))


def _block_rows(b):
    for r in (16, 8, 4, 2, 1):
        if b % r == 0:
            return r
    return 1


@jax.jit
def kernel(x):
    b, d = x.shape
    topk = min(_TOPK, d)
    k1 = topk // 2
    k2 = topk - k1
    r = _block_rows(b)
    idx_bits = max(1, (d - 1).bit_length())
    body = functools.partial(_kcomp_block, k1=k1, k2=k2, factor=_FACTOR,
                             idx_bits=idx_bits)
    return pl.pallas_call(
        body,
        grid=(b // r,),
        in_specs=[pl.BlockSpec((r, d), lambda i: (i, 0))],
        out_specs=pl.BlockSpec((r, d), lambda i: (i, 0)),
        out_shape=jax.ShapeDtypeStruct((b, d), jnp.float32),
    )(x)


# speedup vs baseline: 10.0449x; 1.4738x over previous
import functools

import jax
import jax.numpy as jnp
from jax.experimental import pallas as pl

_FACTOR = 6.26
_TOPK = 128


def _kcomp_block(x_ref, o_ref, *, k1, k2, factor, idx_bits):
    x = x_ref[...]
    r, d = x.shape
    pos = jnp.maximum(x, 0.0)
    neg = jnp.maximum(-x, 0.0)
    pos_bits = jax.lax.bitcast_convert_type(pos, jnp.int32) & 0x7FFFFFFF
    neg_bits = jax.lax.bitcast_convert_type(neg, jnp.int32) & 0x7FFFFFFF

    def count_ge(bits, thr):
        return jnp.sum((bits >= thr).astype(jnp.int32), axis=1, keepdims=True)

    def val_step(i, carry):
        tp, tn = carry
        bit = jnp.int32(1) << (30 - i)
        candp = tp | bit
        candn = tn | bit
        tp = jnp.where(count_ge(pos_bits, candp) >= k1, candp, tp)
        tn = jnp.where(count_ge(neg_bits, candn) >= k2, candn, tn)
        return tp, tn

    zeros = jnp.zeros((r, 1), jnp.int32)
    tp, tn = jax.lax.fori_loop(0, 31, val_step, (zeros, zeros))

    key = (d - 1) - jax.lax.broadcasted_iota(jnp.int32, (r, d), 1)
    eq_p = pos_bits == tp
    eq_n = neg_bits == tn

    def tie_search(_):
        cnt_gt_p = count_ge(pos_bits, tp + 1)
        cnt_gt_n = count_ge(neg_bits, tn + 1)
        need_p = k1 - cnt_gt_p
        need_n = k2 - cnt_gt_n

        def idx_step(i, carry):
            kp, kn = carry
            bit = jnp.int32(1) << (idx_bits - 1 - i)
            candp = kp | bit
            candn = kn | bit
            cp = jnp.sum((eq_p & (key >= candp)).astype(jnp.int32), axis=1,
                         keepdims=True)
            cn = jnp.sum((eq_n & (key >= candn)).astype(jnp.int32), axis=1,
                         keepdims=True)
            kp = jnp.where(cp >= need_p, candp, kp)
            kn = jnp.where(cn >= need_n, candn, kn)
            return kp, kn

        return jax.lax.fori_loop(0, idx_bits, idx_step, (zeros, zeros))

    no_ties = jnp.logical_and(jnp.all(count_ge(pos_bits, tp) == k1),
                              jnp.all(count_ge(neg_bits, tn) == k2))
    kp, kn = jax.lax.cond(no_ties, lambda _: (zeros, zeros), tie_search,
                          operand=None)

    maskp = (pos_bits > tp) | (eq_p & (key >= kp))
    maskn = (neg_bits > tn) | (eq_n & (key >= kn))

    loser_p = jnp.sum(jnp.where(maskp, 0.0, pos), axis=1, keepdims=True)
    loser_n = jnp.sum(jnp.where(maskn, 0.0, neg), axis=1, keepdims=True)
    ptmp = factor * loser_p
    ntmp = factor * loser_n

    o_ref[...] = (jnp.where(maskp, pos + ptmp, 0.0)
                  - jnp.where(maskn, neg + ntmp, 0.0))


def _block_rows(b):
    for r in (16, 8, 4, 2, 1):
        if b % r == 0:
            return r
    return 1


@jax.jit
def kernel(x):
    b, d = x.shape
    topk = min(_TOPK, d)
    k1 = topk // 2
    k2 = topk - k1
    r = _block_rows(b)
    idx_bits = max(1, (d - 1).bit_length())
    body = functools.partial(_kcomp_block, k1=k1, k2=k2, factor=_FACTOR,
                             idx_bits=idx_bits)
    return pl.pallas_call(
        body,
        grid=(b // r,),
        in_specs=[pl.BlockSpec((r, d), lambda i: (i, 0))],
        out_specs=pl.BlockSpec((r, d), lambda i: (i, 0)),
        out_shape=jax.ShapeDtypeStruct((b, d), jnp.float32),
    )(x)
